# Initial kernel scaffold; baseline (speedup 1.0000x reference)
import jax
import jax.numpy as jnp
from jax.experimental import pallas as pl


def kernel(vision, vision_times, proprio, proprio_times, imu, imu_times, target_times, points):
    raise NotImplementedError("write your pallas kernel here")


# speedup vs baseline: 6.1109x; 6.1109x over previous
import functools

import jax
import jax.numpy as jnp
from jax import lax
from jax.experimental import pallas as pl
from jax.experimental.pallas import tpu as pltpu
from jax.experimental.pallas import tpu_sc as plsc

B = 8
TV, DV = 64, 512
TP, DP = 256, 64
TI, DI = 512, 32
TT = 128
NPTS = 262144
GRID = 64
INV_VOX = 16.0
GMIN = -2.0

NW = 32
B_PER_SC = 4
HIST_N = B_PER_SC * GRID * GRID * GRID
PTS_PER_TILE = (B_PER_SC * NPTS) // 16
CHUNK_PTS = 4096
N_CHUNKS = PTS_PER_TILE // CHUNK_PTS
CHUNK_F32 = CHUNK_PTS * 3
GROUPS = CHUNK_PTS // 16
RED_WORDS = HIST_N // 16
RED_CHUNK = 16384


def _sc_voxel_body(points_hbm, zeros_hbm, out_hbm,
                   hist, xbuf, ybuf, zbuf, ibuf, vbuf, rbuf, obuf):
    c = lax.axis_index("c")
    s = lax.axis_index("s")
    wid = c * 16 + s

    pltpu.sync_copy(zeros_hbm.at[pl.ds(s * RED_WORDS, RED_WORDS)],
                    hist.at[pl.ds(s * RED_WORDS, RED_WORDS)])
    plsc.subcore_barrier()

    bbase = (s // 4) * (GRID * GRID * GRID)
    tile_off = c * (B_PER_SC * NPTS) + s * PTS_PER_TILE

    def chunk_body(k, _):
        off = tile_off + k * CHUNK_PTS
        npts_all = B * NPTS
        pltpu.sync_copy(points_hbm.at[pl.ds(off, CHUNK_PTS)], xbuf)
        pltpu.sync_copy(points_hbm.at[pl.ds(npts_all + off, CHUNK_PTS)], ybuf)
        pltpu.sync_copy(points_hbm.at[pl.ds(2 * npts_all + off, CHUNK_PTS)], zbuf)

        def group_body(g, _):
            xs = xbuf[pl.ds(g * 16, 16)]
            ys = ybuf[pl.ds(g * 16, 16)]
            zs = zbuf[pl.ds(g * 16, 16)]
            cx = (xs - GMIN) * INV_VOX
            cy = (ys - GMIN) * INV_VOX
            cz = (zs - GMIN) * INV_VOX
            mn = jnp.minimum(jnp.minimum(cx, cy), cz)
            mx = jnp.maximum(jnp.maximum(cx, cy), cz)
            valid = (mn >= 0.0) & (mx < float(GRID))
            cxi = jnp.minimum(jnp.maximum(cx, 0.0), 63.0).astype(jnp.int32)
            cyi = jnp.minimum(jnp.maximum(cy, 0.0), 63.0).astype(jnp.int32)
            czi = jnp.minimum(jnp.maximum(cz, 0.0), 63.0).astype(jnp.int32)
            flat = bbase + cxi * (GRID * GRID) + cyi * GRID + czi
            val = jnp.where(valid, 1.0, 0.0).astype(jnp.float32)
            r = g // 8
            col = (g % 8) * 16
            ibuf[r, pl.ds(col, 16)] = flat
            vbuf[r, pl.ds(col, 16)] = val
            return 0

        lax.fori_loop(0, GROUPS, group_body, 0)

        def scat_body(j, _):
            pltpu.sync_copy(vbuf.at[j], hist.at[ibuf.at[j]], add=True)
            return 0

        lax.fori_loop(0, GROUPS // 8, scat_body, 0)
        return 0

    lax.fori_loop(0, N_CHUNKS, chunk_body, 0)
    plsc.subcore_barrier()

    acc = (jnp.zeros((16,), jnp.float32), jnp.zeros((16,), jnp.float32),
           jnp.zeros((16,), jnp.float32), jnp.zeros((16,), jnp.float32))
    for q in range(RED_WORDS // RED_CHUNK):
        pltpu.sync_copy(hist.at[pl.ds(s * RED_WORDS + q * RED_CHUNK, RED_CHUNK)],
                        rbuf)

        def red_body(r, a):
            a0, a1, a2, a3 = a
            base = r * 64
            v0 = rbuf[pl.ds(base, 16)]
            v1 = rbuf[pl.ds(base + 16, 16)]
            v2 = rbuf[pl.ds(base + 32, 16)]
            v3 = rbuf[pl.ds(base + 48, 16)]
            one = jnp.float32(1.0)
            zero = jnp.float32(0.0)
            a0 = a0 + jnp.where(v0 > 0.0, one, zero)
            a1 = a1 + jnp.where(v1 > 0.0, one, zero)
            a2 = a2 + jnp.where(v2 > 0.0, one, zero)
            a3 = a3 + jnp.where(v3 > 0.0, one, zero)
            return (a0, a1, a2, a3)

        acc = lax.fori_loop(0, RED_CHUNK // 64, red_body, acc)

    for j in range(4):
        obuf[pl.ds(j * 16, 16)] = acc[j]
    pltpu.sync_copy(obuf, out_hbm.at[wid])


def _sc_partial_summaries(points_flat, zeros):
    mesh = plsc.VectorSubcoreMesh(core_axis_name="c", subcore_axis_name="s")
    kern = functools.partial(
        pl.kernel,
        mesh=mesh,
        out_type=jax.ShapeDtypeStruct((NW, GRID), jnp.float32),
        scratch_types=[
            pltpu.VMEM_SHARED((HIST_N,), jnp.float32),
            pltpu.VMEM((CHUNK_PTS,), jnp.float32),
            pltpu.VMEM((CHUNK_PTS,), jnp.float32),
            pltpu.VMEM((CHUNK_PTS,), jnp.float32),
            pltpu.VMEM((GROUPS // 8, 128), jnp.int32),
            pltpu.VMEM((GROUPS // 8, 128), jnp.float32),
            pltpu.VMEM((RED_CHUNK,), jnp.float32),
            pltpu.VMEM((GRID,), jnp.float32),
        ],
    )(_sc_voxel_body)
    return kern(points_flat, zeros)


def _interp(times, tq, feats):
    T = times.shape[0]
    idx = jnp.sum((times[None, :] < tq[:, None]).astype(jnp.int32), axis=1)
    idx = jnp.clip(idx, 1, T - 1)
    ii = lax.broadcasted_iota(jnp.int32, (TT, T), 1)
    oh0 = (ii == (idx - 1)[:, None]).astype(jnp.float32)
    oh1 = (ii == idx[:, None]).astype(jnp.float32)
    t0 = jnp.sum(oh0 * times[None, :], axis=1)
    t1 = jnp.sum(oh1 * times[None, :], axis=1)
    w = jnp.clip((tq - t0) / (t1 - t0 + 1e-8), 0.0, 1.0)
    M = oh0 * (1.0 - w)[:, None] + oh1 * w[:, None]
    return jnp.dot(M, feats, preferred_element_type=jnp.float32,
                   precision=lax.Precision.HIGHEST)


def _tc_body(vision_ref, vt_ref, proprio_ref, pt_ref, imu_ref, it_ref,
             tt_ref, part_ref, out_ref):
    tq = tt_ref[0, 0, :]
    v = _interp(vt_ref[0, 0, :], tq, vision_ref[0])
    p = _interp(pt_ref[0, 0, :], tq, proprio_ref[0])
    im = _interp(it_ref[0, 0, :], tq, imu_ref[0])
    summ = jnp.sum(part_ref[...], axis=0) * (1.0 / (B * GRID * GRID))
    sb = jnp.broadcast_to(summ[None, :], (TT, GRID))
    out_ref[0] = jnp.concatenate([v, p, im, sb], axis=-1)


def kernel(vision, vision_times, proprio, proprio_times, imu, imu_times,
           target_times, points):
    points_t = points.reshape(B * NPTS, 3).T.reshape(-1)
    zeros = jnp.zeros((HIST_N,), jnp.float32)
    partials = _sc_partial_summaries(points_t, zeros)

    DOUT = DV + DP + DI + GRID
    out = pl.pallas_call(
        _tc_body,
        grid=(B,),
        in_specs=[
            pl.BlockSpec((1, TV, DV), lambda b: (b, 0, 0)),
            pl.BlockSpec((1, 1, TV), lambda b: (b, 0, 0)),
            pl.BlockSpec((1, TP, DP), lambda b: (b, 0, 0)),
            pl.BlockSpec((1, 1, TP), lambda b: (b, 0, 0)),
            pl.BlockSpec((1, TI, DI), lambda b: (b, 0, 0)),
            pl.BlockSpec((1, 1, TI), lambda b: (b, 0, 0)),
            pl.BlockSpec((1, 1, TT), lambda b: (b, 0, 0)),
            pl.BlockSpec((NW, GRID), lambda b: (0, 0)),
        ],
        out_specs=pl.BlockSpec((1, TT, DOUT), lambda b: (b, 0, 0)),
        out_shape=jax.ShapeDtypeStruct((B, TT, DOUT), jnp.float32),
    )(vision, vision_times.reshape(B, 1, TV),
      proprio, proprio_times.reshape(B, 1, TP),
      imu, imu_times.reshape(B, 1, TI),
      target_times.reshape(B, 1, TT), partials)
    return out
